# Initial kernel scaffold; baseline (speedup 1.0000x reference)
import jax
import jax.numpy as jnp
from jax.experimental import pallas as pl


def kernel(q, qs, key_param, value_param):
    raise NotImplementedError("write your pallas kernel here")


# speedup vs baseline: 3.0711x; 3.0711x over previous
import functools

import jax
import jax.numpy as jnp
import numpy as np
from jax import lax
from jax.experimental import pallas as pl
from jax.experimental.pallas import tpu as pltpu
from jax.experimental.pallas import tpu_sc as plsc

N_HEAD = 4
Q_DIM = 8
DEPTH = 8
V_DIM = 8
NV = 2 ** DEPTH
NK = NV - 1

NC = 2
NS = 16
NW = NC * NS
L = 16

CHUNK = 512
EINV = np.float32(np.exp(-1.0))
LN2 = np.float32(np.log(2.0))


def _tree_body(x_hbm, u1_hbm, u2_hbm, c_hbm, uv_hbm, s1_hbm, s2_hbm,
               vt_hbm, svt_hbm, vb_hbm, vs_hbm,
               x_v, u1_v, u2_v, c_v, uv_v, s1_v, s2_v, vt_v, svt_v,
               vb_v, vs_v):
    wid = lax.axis_index("s") * NC + lax.axis_index("c")
    pltpu.sync_copy(s1_hbm, s1_v)
    pltpu.sync_copy(s2_hbm, s2_v)
    pltpu.sync_copy(vt_hbm, vt_v)
    pltpu.sync_copy(svt_hbm, svt_v)

    ixh = lax.broadcasted_iota(jnp.int32, (L,), 0) & (N_HEAD - 1)

    def group_body(g, carry):
        r = g * L
        xs = [x_v[j, pl.ds(r, L)] for j in range(Q_DIM)]
        exs = [jnp.exp(v) for v in xs]
        enxs = [jnp.exp(-v) for v in xs]
        ix = jnp.zeros((L,), jnp.int32)
        T = jnp.zeros((L,), jnp.float32)
        for d in range(DEPTH):
            node = ixh * NK + (2 ** d - 1) + ix
            S = jnp.zeros((L,), jnp.float32)
            for j in range(Q_DIM):
                idx = node + j * (N_HEAD * NK)
                s1 = plsc.load_gather(s1_v, [idx])
                s2 = plsc.load_gather(s2_v, [idx])
                u1 = u1_v[d * Q_DIM + j, pl.ds(r, L)]
                u2 = u2_v[d * Q_DIM + j, pl.ds(r, L)]
                A = jnp.where(u1 < s1, EINV, np.float32(1.0)) + exs[j]
                Bv = jnp.where(u2 < s2, EINV, np.float32(1.0)) + enxs[j]
                S = S + A * Bv / (A + Bv)
            c = c_v[d, pl.ds(r, L)]
            lor = S < c
            ix = 2 * ix + lor.astype(jnp.int32)
            T = T + jnp.where(lor, S, np.float32(1.0) / S)
        node_v = ixh * NV + ix
        for j in range(V_DIM):
            idx = node_v + j * (N_HEAD * NV)
            vw = plsc.load_gather(vt_v, [idx])
            sv = plsc.load_gather(svt_v, [idx])
            uv = uv_v[j, pl.ds(r, L)]
            vb = uv < sv
            vsp = jnp.where(vb, vw, -vw)
            y = jnp.exp(-vsp) + T
            yi = lax.bitcast_convert_type(y, jnp.int32)
            k = (yi >> 23) - 127
            m = lax.bitcast_convert_type((yi & 0x7FFFFF) | 0x3F800000,
                                         jnp.float32)
            adj = m > np.float32(1.5)
            m = jnp.where(adj, m * np.float32(0.5), m)
            kf = (k + adj.astype(jnp.int32)).astype(jnp.float32)
            s = (m - np.float32(1.0)) / (m + np.float32(1.0))
            s2q = s * s
            lnm = np.float32(2.0) * s * (
                np.float32(1.0) + s2q * (
                    np.float32(1.0 / 3) + s2q * (
                        np.float32(1.0 / 5) + s2q * (
                            np.float32(1.0 / 7) + s2q * np.float32(1.0 / 9)))))
            vs_v[j, pl.ds(r, L)] = -(lnm + kf * LN2)
            vb_v[j, pl.ds(r, L)] = jnp.where(vb, np.float32(1.0),
                                             np.float32(0.0))
        return carry

    rows_per_worker = x_hbm.shape[1] // NW
    nchunks = rows_per_worker // CHUNK
    for ch in range(nchunks):
        base = wid * rows_per_worker + ch * CHUNK
        pltpu.sync_copy(x_hbm.at[:, pl.ds(base, CHUNK)], x_v)
        pltpu.sync_copy(u1_hbm.at[:, pl.ds(base, CHUNK)], u1_v)
        pltpu.sync_copy(u2_hbm.at[:, pl.ds(base, CHUNK)], u2_v)
        pltpu.sync_copy(c_hbm.at[:, pl.ds(base, CHUNK)], c_v)
        pltpu.sync_copy(uv_hbm.at[:, pl.ds(base, CHUNK)], uv_v)
        lax.fori_loop(0, CHUNK // L, group_body, jnp.int32(0))
        pltpu.sync_copy(vb_v, vb_hbm.at[:, pl.ds(base, CHUNK)])
        pltpu.sync_copy(vs_v, vs_hbm.at[:, pl.ds(base, CHUNK)])


@jax.jit
def kernel(q, qs, key_param, value_param):
    B, Q = q.shape
    H = N_HEAD
    BH = B * H
    rk = jax.random.key(42)

    qe = jnp.broadcast_to(q[:, None, :], (B, H, Q)).reshape(-1, Q)
    xT = ((2.0 * qe.astype(jnp.float32) - 1.0) * qs).T

    us = jnp.stack([
        jax.random.uniform(jax.random.fold_in(rk, d), (BH, 2 * Q), jnp.float32)
        for d in range(DEPTH)])
    u1T = us[:, :, 0::2].transpose(0, 2, 1).reshape(DEPTH * Q, BH)
    u2T = us[:, :, 1::2].transpose(0, 2, 1).reshape(DEPTH * Q, BH)
    ul = jnp.stack([
        jax.random.uniform(jax.random.fold_in(rk, 100 + d), (BH,), jnp.float32)
        for d in range(DEPTH)])
    cT = (1.0 - ul) / ul
    uvT = jax.random.uniform(jax.random.fold_in(rk, 999), (B, H, V_DIM),
                             jnp.float32).reshape(BH, V_DIM).T

    key_flat = key_param.reshape(-1, 2 * Q)
    sk = jax.nn.sigmoid(key_flat)
    s1t = sk[:, 0::2].T.reshape(-1)
    s2t = sk[:, 1::2].T.reshape(-1)
    value_flat = value_param.reshape(-1, V_DIM)
    vt = value_flat.T.reshape(-1)
    svt = jax.nn.sigmoid(value_flat).T.reshape(-1)

    f32 = jnp.float32
    mesh = plsc.VectorSubcoreMesh(core_axis_name="c", subcore_axis_name="s")
    vbT, vsT = pl.kernel(
        _tree_body,
        out_type=[jax.ShapeDtypeStruct((V_DIM, BH), f32),
                  jax.ShapeDtypeStruct((V_DIM, BH), f32)],
        mesh=mesh,
        compiler_params=pltpu.CompilerParams(needs_layout_passes=False),
        scratch_types=[
            pltpu.VMEM((Q, CHUNK), f32),
            pltpu.VMEM((DEPTH * Q, CHUNK), f32),
            pltpu.VMEM((DEPTH * Q, CHUNK), f32),
            pltpu.VMEM((DEPTH, CHUNK), f32),
            pltpu.VMEM((V_DIM, CHUNK), f32),
            pltpu.VMEM((Q * H * NK,), f32),
            pltpu.VMEM((Q * H * NK,), f32),
            pltpu.VMEM((V_DIM * H * NV,), f32),
            pltpu.VMEM((V_DIM * H * NV,), f32),
            pltpu.VMEM((V_DIM, CHUNK), f32),
            pltpu.VMEM((V_DIM, CHUNK), f32),
        ],
    )(xT, u1T, u2T, cT, uvT, s1t, s2t, vt, svt)

    vb = vbT.T.reshape(B, H, V_DIM).astype(bool)
    value_s = vsT.T.reshape(B, H, V_DIM)
    return (vb, value_s)
